# Initial kernel scaffold; baseline (speedup 1.0000x reference)
import jax
import jax.numpy as jnp
from jax.experimental import pallas as pl


def kernel(candidates, values, batch_idx):
    raise NotImplementedError("write your pallas kernel here")


# speedup vs baseline: 2.4314x; 2.4314x over previous
import functools

import jax
import jax.numpy as jnp
import numpy as np
from jax import lax
from jax.experimental import pallas as pl
from jax.experimental.pallas import tpu as pltpu
from jax.experimental.pallas import tpu_sc as plsc

_N = 32768
_B = 16
_D = 128
_EPS = 0.1

_NW = 32
_CHUNK = _N // _NW
_KSTEPS = _CHUNK // 16


def _np_threefry2x32(k1, k2, x1, x2):
    def rotl(x, d):
        return ((x << np.uint32(d)) | (x >> np.uint32(32 - d))).astype(np.uint32)

    rot = [[13, 15, 26, 6], [17, 29, 16, 24]]
    ks = [np.uint32(k1), np.uint32(k2),
          np.uint32(k1) ^ np.uint32(k2) ^ np.uint32(0x1BD11BDA)]
    x = [(x1 + ks[0]).astype(np.uint32), (x2 + ks[1]).astype(np.uint32)]
    for g in range(5):
        for r in rot[g % 2]:
            x[0] = (x[0] + x[1]).astype(np.uint32)
            x[1] = x[0] ^ rotl(x[1], r)
        x[0] = (x[0] + ks[(g + 1) % 3]).astype(np.uint32)
        x[1] = (x[1] + ks[(g + 2) % 3] + np.uint32(g + 1)).astype(np.uint32)
    return x[0], x[1]


def _np_uniform_key42(shape):
    size = int(np.prod(shape))
    c64 = np.arange(size, dtype=np.uint64)
    c1 = (c64 >> np.uint64(32)).astype(np.uint32)
    c2 = (c64 & np.uint64(0xFFFFFFFF)).astype(np.uint32)
    b1, b2 = _np_threefry2x32(np.uint32(0), np.uint32(42), c1, c2)
    bits = (b1 ^ b2).reshape(shape)
    fb = (bits >> np.uint32(9)) | np.uint32(0x3F800000)
    floats = fb.view(np.float32) - np.float32(1.0)
    tiny = np.float32(np.finfo(np.float32).tiny)
    span = np.float32(np.float32(1.0) - tiny)
    return np.maximum(tiny, (floats * span + tiny).astype(np.float32))


_UNIF = _np_uniform_key42((_B, _N))
_U_TILED = np.ascontiguousarray(
    _UNIF.reshape(_B, _NW, _CHUNK).transpose(1, 0, 2).reshape(_NW, _B * _CHUNK)
)

_mesh = plsc.VectorSubcoreMesh(
    core_axis_name="c", subcore_axis_name="s", num_cores=2, num_subcores=16
)
_NEG_INF = np.float32(-np.inf)


def _wid():
    return lax.axis_index("s") * 2 + lax.axis_index("c")


def _butterfly(x, tmp_ref, lane, op):
    for st in (1, 2, 4, 8):
        tmp_ref[...] = x
        idx = jnp.bitwise_and(lane + st, 15)
        x = op(x, plsc.load_gather(tmp_ref, [idx]))
    return x


@functools.partial(
    pl.kernel,
    out_type=(
        jax.ShapeDtypeStruct((_NW, 16), jnp.int32),
        jax.ShapeDtypeStruct((_NW, 16), jnp.float32),
    ),
    mesh=_mesh,
    compiler_params=pltpu.CompilerParams(needs_layout_passes=False),
    scratch_types=[
        pltpu.VMEM((_CHUNK,), jnp.float32),
        pltpu.VMEM((_CHUNK,), jnp.int32),
        pltpu.VMEM((16,), jnp.int32),
        pltpu.VMEM((16,), jnp.float32),
        pltpu.VMEM((16,), jnp.float32),
    ],
)
def _seg_stats(v_hbm, b_hbm, cnt_hbm, max_hbm, v_v, b_v, cnt_v, max_v, tmp_v):
    w = _wid()
    base = w * _CHUNK
    pltpu.sync_copy(v_hbm.at[pl.ds(base, _CHUNK)], v_v)
    pltpu.sync_copy(b_hbm.at[pl.ds(base, _CHUNK)], b_v)
    lane = lax.iota(jnp.int32, 16)
    cnt_v[...] = jnp.zeros((16,), jnp.int32)
    max_v[...] = jnp.full((16,), _NEG_INF, jnp.float32)
    bminv = _butterfly(b_v[pl.ds(0, 16)].astype(jnp.float32), tmp_v, lane, jnp.minimum)
    bmaxv = _butterfly(
        b_v[pl.ds(_CHUNK - 16, 16)].astype(jnp.float32), tmp_v, lane, jnp.maximum
    )
    bmin = jnp.max(bminv)
    bmax = jnp.max(bmaxv)
    for seg in range(16):

        @pl.when((jnp.float32(seg) >= bmin) & (jnp.float32(seg) <= bmax))
        def _():
            def body(k, carry):
                c, m = carry
                bv = b_v[pl.ds(k * 16, 16)]
                vv = v_v[pl.ds(k * 16, 16)]
                hit = bv == seg
                c = c + jnp.where(hit, jnp.float32(1.0), jnp.float32(0.0))
                m = jnp.maximum(m, jnp.where(hit, vv, _NEG_INF))
                return (c, m)

            c, m = lax.fori_loop(
                0, _KSTEPS, body,
                (jnp.zeros((16,), jnp.float32), jnp.full((16,), _NEG_INF, jnp.float32)),
            )
            c = _butterfly(c, tmp_v, lane, jnp.add)
            m = _butterfly(m, tmp_v, lane, jnp.maximum)
            cnt_v[...] = jnp.where(lane == seg, c.astype(jnp.int32), cnt_v[...])
            max_v[...] = jnp.where(lane == seg, m, max_v[...])

    pltpu.sync_copy(cnt_v, cnt_hbm.at[w])
    pltpu.sync_copy(max_v, max_hbm.at[w])


@functools.partial(
    pl.kernel,
    out_type=(
        jax.ShapeDtypeStruct((_NW, 16), jnp.float32),
        jax.ShapeDtypeStruct((_NW, 16), jnp.int32),
    ),
    mesh=_mesh,
    compiler_params=pltpu.CompilerParams(needs_layout_passes=False),
    scratch_types=[
        pltpu.VMEM((_CHUNK,), jnp.float32),
        pltpu.VMEM((_CHUNK,), jnp.int32),
        pltpu.VMEM((_B * _CHUNK,), jnp.float32),
        pltpu.VMEM((_CHUNK,), jnp.float32),
        pltpu.VMEM((16, 16), jnp.int32),
        pltpu.VMEM((16,), jnp.float32),
        pltpu.VMEM((16,), jnp.float32),
        pltpu.VMEM((16,), jnp.float32),
        pltpu.VMEM((16,), jnp.float32),
        pltpu.VMEM((16,), jnp.int32),
        pltpu.VMEM((16,), jnp.float32),
    ],
)
def _row_argmax(
    v_hbm, b_hbm, g_hbm, s_hbm, vmax_hbm, hilog_hbm, lolog_hbm,
    best_hbm, bidx_hbm,
    v_v, b_v, g_v, bias_v, s_v, vmax_v, hilog_v, lolog_v, best_v, bidx_v, tmp_v,
):
    w = _wid()
    base = w * _CHUNK
    pltpu.sync_copy(v_hbm.at[pl.ds(base, _CHUNK)], v_v)
    pltpu.sync_copy(b_hbm.at[pl.ds(base, _CHUNK)], b_v)
    pltpu.sync_copy(g_hbm.at[w], g_v)
    pltpu.sync_copy(s_hbm, s_v)
    pltpu.sync_copy(vmax_hbm, vmax_v)
    pltpu.sync_copy(hilog_hbm, hilog_v)
    pltpu.sync_copy(lolog_hbm, lolog_v)
    lane = lax.iota(jnp.int32, 16)

    def p1(k, _):
        bv = b_v[pl.ds(k * 16, 16)]
        vv = v_v[pl.ds(k * 16, 16)]
        mx = plsc.load_gather(vmax_v, [bv])
        hb = plsc.load_gather(hilog_v, [bv])
        lb = plsc.load_gather(lolog_v, [bv])
        bias_v[pl.ds(k * 16, 16)] = jnp.where(vv == mx, hb, lb)
        return 0

    lax.fori_loop(0, _KSTEPS, p1, 0)

    bminv = _butterfly(b_v[pl.ds(0, 16)].astype(jnp.float32), tmp_v, lane, jnp.minimum)
    bmaxv = _butterfly(
        b_v[pl.ds(_CHUNK - 16, 16)].astype(jnp.float32), tmp_v, lane, jnp.maximum
    )
    bmin = jnp.max(bminv)
    bmax = jnp.max(bmaxv)
    best_v[...] = jnp.full((16,), _NEG_INF, jnp.float32)
    bidx_v[...] = jnp.zeros((16,), jnp.int32)

    for i in range(16):
        siv = s_v[i]
        si = jnp.max(siv.astype(jnp.float32))

        @pl.when((si >= bmin) & (si <= bmax))
        def _():
            def p2(k, carry):
                rb, ri = carry
                bv = b_v[pl.ds(k * 16, 16)]
                g = g_v[pl.ds(i * _CHUNK + k * 16, 16)]
                bias = bias_v[pl.ds(k * 16, 16)]
                sc = jnp.where(bv == siv, g + bias, _NEG_INF)
                jv = (base + k * 16) + lane
                upd = sc > rb
                return (jnp.where(upd, sc, rb), jnp.where(upd, jv, ri))

            rb, ri = lax.fori_loop(
                0, _KSTEPS, p2,
                (jnp.full((16,), _NEG_INF, jnp.float32), jnp.zeros((16,), jnp.int32)),
            )
            mxv = _butterfly(rb, tmp_v, lane, jnp.maximum)
            rif = jnp.where(rb == mxv, ri.astype(jnp.float32), jnp.float32(1e9))
            mnv = _butterfly(rif, tmp_v, lane, jnp.minimum)
            best_v[...] = jnp.where(lane == i, mxv, best_v[...])
            bidx_v[...] = jnp.where(lane == i, mnv.astype(jnp.int32), bidx_v[...])

    pltpu.sync_copy(best_v, best_hbm.at[w])
    pltpu.sync_copy(bidx_v, bidx_hbm.at[w])


@functools.partial(
    pl.kernel,
    out_type=jax.ShapeDtypeStruct((_B, _D), jnp.float32),
    mesh=_mesh,
    compiler_params=pltpu.CompilerParams(needs_layout_passes=False),
    scratch_types=[
        pltpu.VMEM((_NW, 16), jnp.float32),
        pltpu.VMEM((_NW, 16), jnp.int32),
        pltpu.VMEM((16,), jnp.int32),
        pltpu.VMEM((_B, _D), jnp.float32),
        pltpu.SemaphoreType.DMA,
    ],
)
def _combine_gather(best_hbm, bidx_hbm, cand_hbm, out_hbm,
                    pb_v, pi_v, idx_v, rows_v, sem):
    w = _wid()

    @pl.when(w == 0)
    def _():
        pltpu.sync_copy(best_hbm, pb_v)
        pltpu.sync_copy(bidx_hbm, pi_v)
        acc_b = jnp.full((16,), _NEG_INF, jnp.float32)
        acc_i = jnp.zeros((16,), jnp.int32)
        for t in range(_NW):
            bv = pb_v[t]
            iv = pi_v[t]
            upd = bv > acc_b
            acc_b = jnp.where(upd, bv, acc_b)
            acc_i = jnp.where(upd, iv, acc_i)
        idx_v[...] = acc_i
        pltpu.async_copy(cand_hbm.at[idx_v], rows_v, sem).wait()
        pltpu.sync_copy(rows_v, out_hbm)


def kernel(candidates, values, batch_idx):
    g_tiled = -jnp.log(-jnp.log(jnp.asarray(_U_TILED)))
    cnt_p, max_p = _seg_stats(values, batch_idx)
    cnt = jnp.sum(cnt_p, axis=0)
    vmax = jnp.max(max_p, axis=0)
    q = jnp.float32(_EPS) / (cnt.astype(jnp.float32) - jnp.float32(1.0))
    hi = jnp.float32(1.0) + q
    hi = jnp.where(jnp.isinf(hi), jnp.float32(1.0), hi)
    lo = jnp.where(jnp.isinf(q), jnp.float32(1.0), q)
    hilog = jnp.log(hi)
    lolog = jnp.log(lo)
    present = cnt > 0
    rank = jnp.cumsum(present.astype(jnp.int32)) - 1
    ar = jnp.arange(16, dtype=jnp.int32)
    minp = jnp.min(jnp.where(present, ar, jnp.int32(99)))
    u = jnp.full((16,), minp, jnp.int32)
    u = u.at[jnp.where(present, rank, 16)].set(ar, mode="drop")
    s_flip = u[::-1]
    s_bcast = jnp.broadcast_to(s_flip[:, None], (16, 16))

    best_p, bidx_p = _row_argmax(
        values, batch_idx, g_tiled, s_bcast, vmax, hilog, lolog
    )
    return _combine_gather(best_p, bidx_p, candidates)


# speedup vs baseline: 2.6184x; 1.0769x over previous
import functools

import jax
import jax.numpy as jnp
import numpy as np
from jax import lax
from jax.experimental import pallas as pl
from jax.experimental.pallas import tpu as pltpu
from jax.experimental.pallas import tpu_sc as plsc

_N = 32768
_B = 16
_D = 128
_EPS = 0.1

_NW = 32
_CHUNK = _N // _NW
_KSTEPS = _CHUNK // 16


def _np_threefry2x32(k1, k2, x1, x2):
    def rotl(x, d):
        return ((x << np.uint32(d)) | (x >> np.uint32(32 - d))).astype(np.uint32)

    rot = [[13, 15, 26, 6], [17, 29, 16, 24]]
    ks = [np.uint32(k1), np.uint32(k2),
          np.uint32(k1) ^ np.uint32(k2) ^ np.uint32(0x1BD11BDA)]
    x = [(x1 + ks[0]).astype(np.uint32), (x2 + ks[1]).astype(np.uint32)]
    for g in range(5):
        for r in rot[g % 2]:
            x[0] = (x[0] + x[1]).astype(np.uint32)
            x[1] = x[0] ^ rotl(x[1], r)
        x[0] = (x[0] + ks[(g + 1) % 3]).astype(np.uint32)
        x[1] = (x[1] + ks[(g + 2) % 3] + np.uint32(g + 1)).astype(np.uint32)
    return x[0], x[1]


def _np_uniform_key42(shape):
    size = int(np.prod(shape))
    c64 = np.arange(size, dtype=np.uint64)
    c1 = (c64 >> np.uint64(32)).astype(np.uint32)
    c2 = (c64 & np.uint64(0xFFFFFFFF)).astype(np.uint32)
    b1, b2 = _np_threefry2x32(np.uint32(0), np.uint32(42), c1, c2)
    bits = (b1 ^ b2).reshape(shape)
    fb = (bits >> np.uint32(9)) | np.uint32(0x3F800000)
    floats = fb.view(np.float32) - np.float32(1.0)
    tiny = np.float32(np.finfo(np.float32).tiny)
    span = np.float32(np.float32(1.0) - tiny)
    return np.maximum(tiny, (floats * span + tiny).astype(np.float32))


_UNIF = _np_uniform_key42((_B, _N))
_U_FLAT = np.ascontiguousarray(_UNIF.reshape(_B * _N))

_mesh = plsc.VectorSubcoreMesh(
    core_axis_name="c", subcore_axis_name="s", num_cores=2, num_subcores=16
)
_NEG_INF = np.float32(-np.inf)


def _wid():
    return lax.axis_index("s") * 2 + lax.axis_index("c")


def _butterfly(x, tmp_ref, lane, op):
    for st in (1, 2, 4, 8):
        tmp_ref[...] = x
        idx = jnp.bitwise_and(lane + st, 15)
        x = op(x, plsc.load_gather(tmp_ref, [idx]))
    return x


@functools.partial(
    pl.kernel,
    out_type=(
        jax.ShapeDtypeStruct((_NW, 16), jnp.int32),
        jax.ShapeDtypeStruct((_NW, 16), jnp.float32),
    ),
    mesh=_mesh,
    compiler_params=pltpu.CompilerParams(needs_layout_passes=False),
    scratch_types=[
        pltpu.VMEM((_CHUNK,), jnp.float32),
        pltpu.VMEM((_CHUNK,), jnp.int32),
        pltpu.VMEM((16,), jnp.int32),
        pltpu.VMEM((16,), jnp.float32),
        pltpu.VMEM((16,), jnp.float32),
    ],
)
def _seg_stats(v_hbm, b_hbm, cnt_hbm, max_hbm, v_v, b_v, cnt_v, max_v, tmp_v):
    w = _wid()
    base = w * _CHUNK
    pltpu.sync_copy(v_hbm.at[pl.ds(base, _CHUNK)], v_v)
    pltpu.sync_copy(b_hbm.at[pl.ds(base, _CHUNK)], b_v)
    lane = lax.iota(jnp.int32, 16)
    cnt_v[...] = jnp.zeros((16,), jnp.int32)
    max_v[...] = jnp.full((16,), _NEG_INF, jnp.float32)
    bminv = _butterfly(b_v[pl.ds(0, 16)].astype(jnp.float32), tmp_v, lane, jnp.minimum)
    bmaxv = _butterfly(
        b_v[pl.ds(_CHUNK - 16, 16)].astype(jnp.float32), tmp_v, lane, jnp.maximum
    )
    bmin = jnp.max(bminv)
    bmax = jnp.max(bmaxv)
    for seg in range(16):

        @pl.when((jnp.float32(seg) >= bmin) & (jnp.float32(seg) <= bmax))
        def _():
            def body(k, carry):
                c, m = carry
                bv = b_v[pl.ds(k * 16, 16)]
                vv = v_v[pl.ds(k * 16, 16)]
                hit = bv == seg
                c = c + jnp.where(hit, jnp.float32(1.0), jnp.float32(0.0))
                m = jnp.maximum(m, jnp.where(hit, vv, _NEG_INF))
                return (c, m)

            c, m = lax.fori_loop(
                0, _KSTEPS, body,
                (jnp.zeros((16,), jnp.float32), jnp.full((16,), _NEG_INF, jnp.float32)),
            )
            c = _butterfly(c, tmp_v, lane, jnp.add)
            m = _butterfly(m, tmp_v, lane, jnp.maximum)
            cnt_v[...] = jnp.where(lane == seg, c.astype(jnp.int32), cnt_v[...])
            max_v[...] = jnp.where(lane == seg, m, max_v[...])

    pltpu.sync_copy(cnt_v, cnt_hbm.at[w])
    pltpu.sync_copy(max_v, max_hbm.at[w])


_KS2 = (_N // 16) // 16
_CH2 = _N // 16


@functools.partial(
    pl.kernel,
    out_type=(
        jax.ShapeDtypeStruct((_B, _D), jnp.float32),
        jax.ShapeDtypeStruct((16, 16), jnp.float32),
        jax.ShapeDtypeStruct((16, 16), jnp.int32),
    ),
    mesh=_mesh,
    compiler_params=pltpu.CompilerParams(needs_layout_passes=False),
    scratch_types=[
        pltpu.VMEM((_CH2,), jnp.float32),
        pltpu.VMEM((_CH2,), jnp.int32),
        pltpu.VMEM((_B * _CH2,), jnp.float32),
        pltpu.VMEM((_CH2,), jnp.float32),
        pltpu.VMEM((16, 16), jnp.int32),
        pltpu.VMEM((16,), jnp.float32),
        pltpu.VMEM((16,), jnp.float32),
        pltpu.VMEM((16,), jnp.float32),
        pltpu.VMEM((16,), jnp.float32),
        pltpu.VMEM((16,), jnp.int32),
        pltpu.VMEM((16,), jnp.float32),
        pltpu.VMEM((16, 16), jnp.float32),
        pltpu.VMEM((16, 16), jnp.int32),
        pltpu.VMEM((16,), jnp.int32),
        pltpu.VMEM((_B, _D), jnp.float32),
        pltpu.SemaphoreType.DMA,
        pltpu.SemaphoreType.DMA,
    ],
)
def _row_argmax_gather(
    v_hbm, b_hbm, g_hbm, s_hbm, vmax_hbm, hilog_hbm, lolog_hbm, cand_hbm,
    out_hbm, best_hbm, bidx_hbm,
    v_v, b_v, g_v, bias_v, s_v, vmax_v, hilog_v, lolog_v, best_v, bidx_v,
    tmp_v, pb_v, pi_v, idx_v, rows_v, gsem, csem,
):
    cid = lax.axis_index("c")
    sid = lax.axis_index("s")

    @pl.when(cid == 0)
    def _():
        base = sid * _CH2
        pltpu.sync_copy(v_hbm.at[pl.ds(base, _CH2)], v_v)
        pltpu.sync_copy(b_hbm.at[pl.ds(base, _CH2)], b_v)
        pltpu.sync_copy(s_hbm, s_v)
        pltpu.sync_copy(vmax_hbm, vmax_v)
        pltpu.sync_copy(hilog_hbm, hilog_v)
        pltpu.sync_copy(lolog_hbm, lolog_v)
        lane = lax.iota(jnp.int32, 16)

        bminv = _butterfly(
            b_v[pl.ds(0, 16)].astype(jnp.float32), tmp_v, lane, jnp.minimum
        )
        bmaxv = _butterfly(
            b_v[pl.ds(_CH2 - 16, 16)].astype(jnp.float32), tmp_v, lane, jnp.maximum
        )
        bmin = jnp.max(bminv)
        bmax = jnp.max(bmaxv)

        for i in range(16):
            siv = s_v[i]
            si = jnp.max(siv.astype(jnp.float32))

            @pl.when((si >= bmin) & (si <= bmax))
            def _():
                pltpu.async_copy(
                    g_hbm.at[pl.ds(i * _N + base, _CH2)],
                    g_v.at[pl.ds(i * _CH2, _CH2)],
                    gsem,
                )

        def p1(k, _):
            bv = b_v[pl.ds(k * 16, 16)]
            vv = v_v[pl.ds(k * 16, 16)]
            mx = plsc.load_gather(vmax_v, [bv])
            hb = plsc.load_gather(hilog_v, [bv])
            lb = plsc.load_gather(lolog_v, [bv])
            bias_v[pl.ds(k * 16, 16)] = jnp.where(vv == mx, hb, lb)
            return 0

        lax.fori_loop(0, _KS2, p1, 0)

        for i in range(16):
            siv = s_v[i]
            si = jnp.max(siv.astype(jnp.float32))

            @pl.when((si >= bmin) & (si <= bmax))
            def _():
                pltpu.make_async_copy(
                    g_hbm.at[pl.ds(i * _N + base, _CH2)],
                    g_v.at[pl.ds(i * _CH2, _CH2)],
                    gsem,
                ).wait()

        best_v[...] = jnp.full((16,), _NEG_INF, jnp.float32)
        bidx_v[...] = jnp.zeros((16,), jnp.int32)

        for i in range(16):
            siv = s_v[i]
            si = jnp.max(siv.astype(jnp.float32))

            @pl.when((si >= bmin) & (si <= bmax))
            def _():
                def p2(k, carry):
                    rb, ri = carry
                    bv = b_v[pl.ds(k * 16, 16)]
                    g = g_v[pl.ds(i * _CH2 + k * 16, 16)]
                    bias = bias_v[pl.ds(k * 16, 16)]
                    sc = jnp.where(bv == siv, g + bias, _NEG_INF)
                    jv = (base + k * 16) + lane
                    upd = sc > rb
                    return (jnp.where(upd, sc, rb), jnp.where(upd, jv, ri))

                rb, ri = lax.fori_loop(
                    0, _KS2, p2,
                    (jnp.full((16,), _NEG_INF, jnp.float32),
                     jnp.zeros((16,), jnp.int32)),
                )
                mxv = _butterfly(rb, tmp_v, lane, jnp.maximum)
                rif = jnp.where(
                    rb == mxv, ri.astype(jnp.float32), jnp.float32(1e9)
                )
                mnv = _butterfly(rif, tmp_v, lane, jnp.minimum)
                best_v[...] = jnp.where(lane == i, mxv, best_v[...])
                bidx_v[...] = jnp.where(
                    lane == i, mnv.astype(jnp.int32), bidx_v[...]
                )

        pltpu.sync_copy(best_v, best_hbm.at[sid])
        pltpu.sync_copy(bidx_v, bidx_hbm.at[sid])
        plsc.subcore_barrier()

        @pl.when(sid == 0)
        def _():
            pltpu.sync_copy(best_hbm, pb_v)
            pltpu.sync_copy(bidx_hbm, pi_v)
            acc_b = jnp.full((16,), _NEG_INF, jnp.float32)
            acc_i = jnp.zeros((16,), jnp.int32)
            for t in range(16):
                bv = pb_v[t]
                iv = pi_v[t]
                upd = bv > acc_b
                acc_b = jnp.where(upd, bv, acc_b)
                acc_i = jnp.where(upd, iv, acc_i)
            idx_v[...] = acc_i
            pltpu.async_copy(cand_hbm.at[idx_v], rows_v, csem).wait()
            pltpu.sync_copy(rows_v, out_hbm)


def kernel(candidates, values, batch_idx):
    g_flat = -jnp.log(-jnp.log(jnp.asarray(_U_FLAT)))
    cnt_p, max_p = _seg_stats(values, batch_idx)
    cnt = jnp.sum(cnt_p, axis=0)
    vmax = jnp.max(max_p, axis=0)
    q = jnp.float32(_EPS) / (cnt.astype(jnp.float32) - jnp.float32(1.0))
    hi = jnp.float32(1.0) + q
    hi = jnp.where(jnp.isinf(hi), jnp.float32(1.0), hi)
    lo = jnp.where(jnp.isinf(q), jnp.float32(1.0), q)
    hilog = jnp.log(hi)
    lolog = jnp.log(lo)
    present = cnt > 0
    rank = jnp.cumsum(present.astype(jnp.int32)) - 1
    ar = jnp.arange(16, dtype=jnp.int32)
    minp = jnp.min(jnp.where(present, ar, jnp.int32(99)))
    u = jnp.full((16,), minp, jnp.int32)
    u = u.at[jnp.where(present, rank, 16)].set(ar, mode="drop")
    s_flip = u[::-1]
    s_bcast = jnp.broadcast_to(s_flip[:, None], (16, 16))

    out, _, _ = _row_argmax_gather(
        values, batch_idx, g_flat, s_bcast, vmax, hilog, lolog, candidates
    )
    return out


# speedup vs baseline: 3.2496x; 1.2411x over previous
import functools

import jax
import jax.numpy as jnp
import numpy as np
from jax import lax
from jax.experimental import pallas as pl
from jax.experimental.pallas import tpu as pltpu
from jax.experimental.pallas import tpu_sc as plsc

_N = 32768
_B = 16
_D = 128
_EPS = 0.1


def _np_threefry2x32(k1, k2, x1, x2):
    def rotl(x, d):
        return ((x << np.uint32(d)) | (x >> np.uint32(32 - d))).astype(np.uint32)

    rot = [[13, 15, 26, 6], [17, 29, 16, 24]]
    ks = [np.uint32(k1), np.uint32(k2),
          np.uint32(k1) ^ np.uint32(k2) ^ np.uint32(0x1BD11BDA)]
    x = [(x1 + ks[0]).astype(np.uint32), (x2 + ks[1]).astype(np.uint32)]
    for g in range(5):
        for r in rot[g % 2]:
            x[0] = (x[0] + x[1]).astype(np.uint32)
            x[1] = x[0] ^ rotl(x[1], r)
        x[0] = (x[0] + ks[(g + 1) % 3]).astype(np.uint32)
        x[1] = (x[1] + ks[(g + 2) % 3] + np.uint32(g + 1)).astype(np.uint32)
    return x[0], x[1]


def _np_uniform_key42(shape):
    size = int(np.prod(shape))
    c64 = np.arange(size, dtype=np.uint64)
    c1 = (c64 >> np.uint64(32)).astype(np.uint32)
    c2 = (c64 & np.uint64(0xFFFFFFFF)).astype(np.uint32)
    b1, b2 = _np_threefry2x32(np.uint32(0), np.uint32(42), c1, c2)
    bits = (b1 ^ b2).reshape(shape)
    fb = (bits >> np.uint32(9)) | np.uint32(0x3F800000)
    floats = fb.view(np.float32) - np.float32(1.0)
    tiny = np.float32(np.finfo(np.float32).tiny)
    span = np.float32(np.float32(1.0) - tiny)
    return np.maximum(tiny, (floats * span + tiny).astype(np.float32))


_UNIF = _np_uniform_key42((_B, _N))
_U_FLAT = np.ascontiguousarray(_UNIF.reshape(_B * _N))

_mesh = plsc.VectorSubcoreMesh(
    core_axis_name="c", subcore_axis_name="s", num_cores=2, num_subcores=16
)
_NEG_INF = np.float32(-np.inf)


def _butterfly(x, tmp_ref, lane, op):
    for st in (1, 2, 4, 8):
        tmp_ref[...] = x
        idx = jnp.bitwise_and(lane + st, 15)
        x = op(x, plsc.load_gather(tmp_ref, [idx]))
    return x


def _psum(x, tmp_ref, lane):
    for st in (1, 2, 4, 8):
        tmp_ref[...] = x
        idx = jnp.maximum(lane - st, 0)
        sh = plsc.load_gather(tmp_ref, [idx])
        x = x + jnp.where(lane >= st, sh, jnp.float32(0.0))
    return x


_CH2 = _N // 16
_KS2 = _CH2 // 16


@functools.partial(
    pl.kernel,
    out_type=jax.ShapeDtypeStruct((_B, _D), jnp.float32),
    mesh=_mesh,
    compiler_params=pltpu.CompilerParams(needs_layout_passes=False),
    scratch_types=[
        pltpu.VMEM((_CH2,), jnp.float32),
        pltpu.VMEM((_CH2,), jnp.int32),
        pltpu.VMEM((8 * _CH2,), jnp.float32),
        pltpu.VMEM((_CH2,), jnp.float32),
        pltpu.VMEM((16,), jnp.float32),
        pltpu.VMEM((16,), jnp.float32),
        pltpu.VMEM((16,), jnp.float32),
        pltpu.VMEM((16,), jnp.float32),
        pltpu.VMEM((16,), jnp.int32),
        pltpu.VMEM((16,), jnp.float32),
        pltpu.VMEM((16, 16), jnp.float32),
        pltpu.VMEM((16, 16), jnp.float32),
        pltpu.VMEM((16, 16), jnp.int32),
        pltpu.VMEM((16,), jnp.int32),
        pltpu.VMEM((16,), jnp.int32),
        pltpu.VMEM((16,), jnp.int32),
        pltpu.VMEM((_B, _D), jnp.float32),
        pltpu.HBM((16, 16), jnp.float32),
        pltpu.HBM((16, 16), jnp.float32),
        pltpu.HBM((16, 16), jnp.float32),
        pltpu.HBM((16, 16), jnp.int32),
        pltpu.SemaphoreType.DMA,
        pltpu.SemaphoreType.DMA,
        pltpu.SemaphoreType.DMA,
    ],
)
def _gaqn_full(
    v_hbm, b_hbm, g_hbm, hit_hbm, lot_hbm, cand_hbm,
    out_hbm,
    v_v, b_v, g_v, bias_v, vmax_v, hilog_v, lolog_v, best_v, bidx_v,
    tmp_v, ex_a, ex_b, pi_v, cnt_i, u_ref, idx_v, rows_v,
    sh_cnt, sh_max, sh_best, sh_idx,
    gsem, tsem, csem,
):
    cid = lax.axis_index("c")
    sid = lax.axis_index("s")

    @pl.when(cid == 0)
    def _():
        base = sid * _CH2
        pltpu.sync_copy(v_hbm.at[pl.ds(base, _CH2)], v_v)
        pltpu.sync_copy(b_hbm.at[pl.ds(base, _CH2)], b_v)
        lane = lax.iota(jnp.int32, 16)
        lanef = lane.astype(jnp.float32)

        bminv = _butterfly(
            b_v[pl.ds(0, 16)].astype(jnp.float32), tmp_v, lane, jnp.minimum
        )
        bmaxv = _butterfly(
            b_v[pl.ds(_CH2 - 16, 16)].astype(jnp.float32), tmp_v, lane, jnp.maximum
        )
        bmin = jnp.max(bminv)
        bmax = jnp.max(bmaxv)

        hilog_v[...] = jnp.zeros((16,), jnp.float32)
        lolog_v[...] = jnp.full((16,), _NEG_INF, jnp.float32)
        for seg in range(16):

            @pl.when((jnp.float32(seg) >= bmin) & (jnp.float32(seg) <= bmax))
            def _():
                def body(k, carry):
                    c, m = carry
                    bv = b_v[pl.ds(k * 16, 16)]
                    vv = v_v[pl.ds(k * 16, 16)]
                    hit = bv == seg
                    c = c + jnp.where(hit, jnp.float32(1.0), jnp.float32(0.0))
                    m = jnp.maximum(m, jnp.where(hit, vv, _NEG_INF))
                    return (c, m)

                c, m = lax.fori_loop(
                    0, _KS2, body,
                    (jnp.zeros((16,), jnp.float32),
                     jnp.full((16,), _NEG_INF, jnp.float32)),
                )
                c = _butterfly(c, tmp_v, lane, jnp.add)
                m = _butterfly(m, tmp_v, lane, jnp.maximum)
                hilog_v[...] = jnp.where(lane == seg, c, hilog_v[...])
                lolog_v[...] = jnp.where(lane == seg, m, lolog_v[...])

        pltpu.sync_copy(hilog_v, sh_cnt.at[sid])
        pltpu.sync_copy(lolog_v, sh_max.at[sid])
        plsc.subcore_barrier()

        pltpu.sync_copy(sh_cnt, ex_a)
        pltpu.sync_copy(sh_max, ex_b)
        cntf = jnp.zeros((16,), jnp.float32)
        vmax = jnp.full((16,), _NEG_INF, jnp.float32)
        for t in range(16):
            cntf = cntf + ex_a[t]
            vmax = jnp.maximum(vmax, ex_b[t])
        vmax_v[...] = vmax

        cnt_i[...] = cntf.astype(jnp.int32)
        pltpu.async_copy(hit_hbm.at[cnt_i], hilog_v, tsem)
        pltpu.async_copy(lot_hbm.at[cnt_i], lolog_v, tsem).wait()
        pltpu.make_async_copy(hit_hbm.at[cnt_i], hilog_v, tsem).wait()

        present = cntf > jnp.float32(0.0)
        presf = jnp.where(present, jnp.float32(1.0), jnp.float32(0.0))
        rank = (_psum(presf, tmp_v, lane) - jnp.float32(1.0)).astype(jnp.int32)
        minpv = _butterfly(
            jnp.where(present, lanef, jnp.float32(99.0)), tmp_v, lane,
            jnp.minimum,
        )
        u_ref[...] = minpv.astype(jnp.int32)
        plsc.store_scatter(
            u_ref, [jnp.where(present, rank, 0)], lane, mask=present
        )
        s_flip = lax.rev(u_ref[...], (0,)).astype(jnp.float32)

        sivs = []
        for i in range(16):
            siv_f = _butterfly(
                jnp.where(lane == i, s_flip, jnp.float32(-1e9)), tmp_v, lane,
                jnp.maximum,
            )
            sivs.append(siv_f)

        best_v[...] = jnp.full((16,), _NEG_INF, jnp.float32)
        bidx_v[...] = jnp.zeros((16,), jnp.int32)

        for wave in range(2):
            for i in range(8 * wave, 8 * wave + 8):
                slot = i % 8
                si = jnp.max(sivs[i])

                @pl.when((si >= bmin) & (si <= bmax))
                def _():
                    pltpu.async_copy(
                        g_hbm.at[pl.ds(i * _N + base, _CH2)],
                        g_v.at[pl.ds(slot * _CH2, _CH2)],
                        gsem,
                    )

            if wave == 0:
                def p1(k, _):
                    bv = b_v[pl.ds(k * 16, 16)]
                    vv = v_v[pl.ds(k * 16, 16)]
                    mx = plsc.load_gather(vmax_v, [bv])
                    hb = plsc.load_gather(hilog_v, [bv])
                    lb = plsc.load_gather(lolog_v, [bv])
                    bias_v[pl.ds(k * 16, 16)] = jnp.where(vv == mx, hb, lb)
                    return 0

                lax.fori_loop(0, _KS2, p1, 0)

            for i in range(8 * wave, 8 * wave + 8):
                slot = i % 8
                si = jnp.max(sivs[i])

                @pl.when((si >= bmin) & (si <= bmax))
                def _():
                    pltpu.make_async_copy(
                        g_hbm.at[pl.ds(i * _N + base, _CH2)],
                        g_v.at[pl.ds(slot * _CH2, _CH2)],
                        gsem,
                    ).wait()

            for i in range(8 * wave, 8 * wave + 8):
                slot = i % 8
                siv = sivs[i].astype(jnp.int32)
                si = jnp.max(sivs[i])

                @pl.when((si >= bmin) & (si <= bmax))
                def _():
                    def p2(k, carry):
                        rb, ri = carry
                        bv = b_v[pl.ds(k * 16, 16)]
                        g = g_v[pl.ds(slot * _CH2 + k * 16, 16)]
                        bias = bias_v[pl.ds(k * 16, 16)]
                        sc = jnp.where(bv == siv, g + bias, _NEG_INF)
                        jv = (base + k * 16) + lane
                        upd = sc > rb
                        return (jnp.where(upd, sc, rb), jnp.where(upd, jv, ri))

                    rb, ri = lax.fori_loop(
                        0, _KS2, p2,
                        (jnp.full((16,), _NEG_INF, jnp.float32),
                         jnp.zeros((16,), jnp.int32)),
                    )
                    mxv = _butterfly(rb, tmp_v, lane, jnp.maximum)
                    rif = jnp.where(
                        rb == mxv, ri.astype(jnp.float32), jnp.float32(1e9)
                    )
                    mnv = _butterfly(rif, tmp_v, lane, jnp.minimum)
                    best_v[...] = jnp.where(lane == i, mxv, best_v[...])
                    bidx_v[...] = jnp.where(
                        lane == i, mnv.astype(jnp.int32), bidx_v[...]
                    )

        pltpu.sync_copy(best_v, sh_best.at[sid])
        pltpu.sync_copy(bidx_v, sh_idx.at[sid])
        plsc.subcore_barrier()

        @pl.when(sid == 0)
        def _():
            pltpu.sync_copy(sh_best, ex_a)
            pltpu.sync_copy(sh_idx, pi_v)
            acc_b = jnp.full((16,), _NEG_INF, jnp.float32)
            acc_i = jnp.zeros((16,), jnp.int32)
            for t in range(16):
                bv = ex_a[t]
                iv = pi_v[t]
                upd = bv > acc_b
                acc_b = jnp.where(upd, bv, acc_b)
                acc_i = jnp.where(upd, iv, acc_i)
            idx_v[...] = acc_i
            pltpu.async_copy(cand_hbm.at[idx_v], rows_v, csem).wait()
            pltpu.sync_copy(rows_v, out_hbm)


def kernel(candidates, values, batch_idx):
    g_flat = -jnp.log(-jnp.log(jnp.asarray(_U_FLAT)))
    carr = jnp.arange(_N + 1, dtype=jnp.int32).astype(jnp.float32)
    q = jnp.float32(_EPS) / (carr - jnp.float32(1.0))
    hi = jnp.float32(1.0) + q
    hi = jnp.where(jnp.isinf(hi), jnp.float32(1.0), hi)
    lo = jnp.where(jnp.isinf(q), jnp.float32(1.0), q)
    hi_tab = jnp.log(hi)
    lo_tab = jnp.log(lo)
    return _gaqn_full(values, batch_idx, g_flat, hi_tab, lo_tab, candidates)
